# speedup vs baseline: 2.3924x; 2.3924x over previous
import functools
import math

import jax
import jax.numpy as jnp
from jax.experimental import pallas as pl
from jax.experimental.pallas import tpu as pltpu

_DIM = 1024
_FF = 2048
_E = 16
_COEFF = 0.01


def _routing_body(x_ref, wg_ref, bg_ref, w_ref, lb_ref, *, n_exp):
    x = x_ref[:]
    logits = jax.lax.dot_general(
        x, wg_ref[:], (((1,), (1,)), ((), ())),
        preferred_element_type=jnp.float32,
    ) + bg_ref[:]
    m = jnp.max(logits, axis=-1, keepdims=True)
    ex = jnp.exp(logits - m)
    probs = ex / jnp.sum(ex, axis=-1, keepdims=True)

    lane = jax.lax.broadcasted_iota(jnp.int32, probs.shape, 1)
    m1 = jnp.max(probs, axis=-1, keepdims=True)
    idx1 = jnp.min(jnp.where(probs == m1, lane, n_exp), axis=-1, keepdims=True)
    sel1 = lane == idx1
    probs2 = jnp.where(sel1, -1.0, probs)
    m2 = jnp.max(probs2, axis=-1, keepdims=True)
    idx2 = jnp.min(jnp.where(probs2 == m2, lane, n_exp), axis=-1, keepdims=True)
    sel2 = lane == idx2

    denom = m1 + m2 + 1e-6
    w = jnp.where(sel1, m1 / denom, 0.0) + jnp.where(sel2, m2 / denom, 0.0)
    w_ref[:] = w

    ntok = probs.shape[0]
    frac = jnp.sum((sel1 | sel2).astype(jnp.float32), axis=0) / ntok
    meanp = jnp.sum(probs, axis=0) / ntok
    lb_ref[0, 0] = _COEFF * n_exp * jnp.sum(frac * meanp)


def _expert_body(x_ref, w1a_ref, w1g_ref, b1a_ref, b1g_ref, w2_ref, b2_ref,
                 wd_ref, out_ref, *, n_exp, fb):
    e = pl.program_id(0)
    f = pl.program_id(1)

    @pl.when((e == 0) & (f == 0))
    def _():
        out_ref[:] = jnp.zeros_like(out_ref)

    xb = x_ref[:]
    w1a = w1a_ref[0].astype(jnp.bfloat16)
    w1g = w1g_ref[0].astype(jnp.bfloat16)
    a = jax.lax.dot_general(
        xb, w1a, (((1,), (1,)), ((), ())),
        preferred_element_type=jnp.float32,
    ) + b1a_ref[0]
    g = jax.lax.dot_general(
        xb, w1g, (((1,), (1,)), ((), ())),
        preferred_element_type=jnp.float32,
    ) + b1g_ref[0]
    act = a * (0.5 * g * (1.0 + jax.lax.erf(g * (1.0 / math.sqrt(2.0)))))

    wd = wd_ref[:]
    sel = (jax.lax.broadcasted_iota(jnp.int32, (n_exp, fb), 0) == e).astype(
        jnp.float32)
    wb = jax.lax.dot_general(
        wd, sel, (((1,), (0,)), ((), ())), preferred_element_type=jnp.float32)

    actw = (act * wb).astype(jnp.bfloat16)
    contrib = jax.lax.dot_general(
        actw, w2_ref[0].astype(jnp.bfloat16), (((1,), (1,)), ((), ())),
        preferred_element_type=jnp.float32,
    )
    out_ref[:] += contrib

    @pl.when(f == 0)
    def _():
        sel_o = (jax.lax.broadcasted_iota(jnp.int32, (n_exp, out_ref.shape[1]),
                                          0) == e).astype(jnp.float32)
        wb_o = jax.lax.dot_general(
            wd, sel_o, (((1,), (0,)), ((), ())),
            preferred_element_type=jnp.float32)
        out_ref[:] += wb_o * b2_ref[0]


@functools.partial(jax.jit, static_argnames=("fb",))
def _moe(x, W1, b1, W2, b2, Wg, bg, fb=512):
    bsz, seq, dim = x.shape
    n_tok = bsz * seq
    n_exp, ff2, _ = W1.shape
    ff = ff2 // 2
    nf = ff // fb

    xr = x.reshape(n_tok, dim)
    w_dense, lb = pl.pallas_call(
        functools.partial(_routing_body, n_exp=n_exp),
        grid=(1,),
        in_specs=[
            pl.BlockSpec((n_tok, dim), lambda i: (0, 0)),
            pl.BlockSpec((n_exp, dim), lambda i: (0, 0)),
            pl.BlockSpec((1, n_exp), lambda i: (0, 0)),
        ],
        out_specs=[
            pl.BlockSpec((n_tok, n_exp), lambda i: (0, 0)),
            pl.BlockSpec(memory_space=pltpu.SMEM),
        ],
        out_shape=[
            jax.ShapeDtypeStruct((n_tok, n_exp), jnp.float32),
            jax.ShapeDtypeStruct((1, 1), jnp.float32),
        ],
    )(xr, Wg, bg.reshape(1, n_exp))

    xb = xr.astype(jnp.bfloat16)
    b1r = b1.reshape(n_exp, 1, ff2)
    b2r = b2.reshape(n_exp, 1, dim)

    out = pl.pallas_call(
        functools.partial(_expert_body, n_exp=n_exp, fb=fb),
        grid=(n_exp, nf),
        in_specs=[
            pl.BlockSpec((n_tok, dim), lambda e, f: (0, 0)),
            pl.BlockSpec((1, fb, dim), lambda e, f: (e, f, 0)),
            pl.BlockSpec((1, fb, dim), lambda e, f, _nf=nf: (e, _nf + f, 0)),
            pl.BlockSpec((1, 1, fb), lambda e, f: (e, 0, f)),
            pl.BlockSpec((1, 1, fb), lambda e, f, _nf=nf: (e, 0, _nf + f)),
            pl.BlockSpec((1, dim, fb), lambda e, f: (e, 0, f)),
            pl.BlockSpec((1, 1, dim), lambda e, f: (e, 0, 0)),
            pl.BlockSpec((n_tok, n_exp), lambda e, f: (0, 0)),
        ],
        out_specs=pl.BlockSpec((n_tok, dim), lambda e, f: (0, 0)),
        out_shape=jax.ShapeDtypeStruct((n_tok, dim), jnp.float32),
        compiler_params=pltpu.CompilerParams(
            dimension_semantics=("arbitrary", "arbitrary"),
        ),
    )(xb, W1, W1, b1r, b1r, W2, b2r, w_dense)

    return out.reshape(bsz, seq, dim), lb.reshape(())


def kernel(x, W1, b1, W2, b2, Wg, bg):
    return _moe(x, W1, b1, W2, b2, Wg, bg)
